# baseline (speedup 1.0000x reference)
import functools

import jax
import jax.numpy as jnp
from jax import lax
from jax.experimental import pallas as pl
from jax.experimental.pallas import tpu as pltpu
from jax.experimental.pallas import tpu_sc as plsc

NN = 10000
DF = 128
NC = 2
NS = 16
NW = NC * NS
PADN = 10112
RPT = PADN // NS
KE = 128
EPAD = 331776
ET0 = 11520
ET1 = EPAD // NS - ET0
STEPS0 = ET0 // KE
STEPS1 = ET1 // KE
ETDEG = EPAD // NW
STEPSDEG = ETDEG // KE
NBUF = 3

_MESH = plsc.VectorSubcoreMesh(
    core_axis_name="c", subcore_axis_name="s", num_cores=NC, num_subcores=NS)


def _deg_body(rc_hbm, out_hbm, colv, degv):
    c = lax.axis_index("c")
    s = lax.axis_index("s")
    wid = c * NS + s
    pltpu.sync_copy(rc_hbm.at[wid], colv)

    zeros16 = jnp.zeros((16,), jnp.float32)
    ones16 = jnp.ones((16,), jnp.float32)

    def zero_step(i, _):
        degv[pl.ds(i * 16, 16)] = zeros16
        return 0

    lax.fori_loop(0, PADN // 16, zero_step, 0)

    def acc_step(r, _):
        for q in range(KE // 16):
            idx = lax.shift_right_logical(colv[r, pl.ds(q * 16, 16)], 16)
            plsc.addupdate_scatter(degv, [idx], ones16)
        return 0

    lax.fori_loop(0, STEPSDEG, acc_step, 0)
    pltpu.sync_copy(degv, out_hbm.at[wid])


_deg_kernel = functools.partial(
    pl.kernel,
    out_type=jax.ShapeDtypeStruct((NW, PADN), jnp.float32),
    mesh=_MESH,
    scratch_types=[
        pltpu.VMEM((STEPSDEG, KE), jnp.int32),
        pltpu.VMEM((PADN,), jnp.float32),
    ],
    compiler_params=pltpu.CompilerParams(needs_layout_passes=False),
)(_deg_body)


def _agg_body(hs_hbm, rc_hbm, out_hbm, rcb, colb, gbuf, acc,
              isem, gsem, ssem):
    c = lax.axis_index("c")
    s = lax.axis_index("s")
    steps = jnp.where(c == 0, STEPS0, STEPS1)
    base = jnp.where(c == 0, s * ET0, NS * ET0 + s * ET1)

    zeros16 = jnp.zeros((16,), jnp.float32)

    def zrow(i, _):
        for j in range(DF // 16):
            gbuf[0, i, pl.ds(j * 16, 16)] = zeros16
        return 0

    lax.fori_loop(0, KE, zrow, 0)
    for z in range((RPT + KE - 1) // KE):
        n = min(KE, RPT - z * KE)
        pltpu.sync_copy(gbuf.at[0, pl.ds(0, n)],
                        acc.at[pl.ds(s * RPT + z * KE, n)])
    plsc.subcore_barrier()

    lomask = jnp.full((16,), 65535, jnp.int32)

    def istart(j, b):
        off = base + j * KE
        pltpu.async_copy(rc_hbm.at[pl.ds(off, KE)], rcb.at[b], isem)

    def iwait(b):
        pltpu.make_async_copy(rc_hbm.at[pl.ds(0, KE)], rcb.at[b],
                              isem).wait()

    def split(b):
        for q in range(KE // 16):
            rc = rcb[b, pl.ds(q * 16, 16)]
            colb[b, pl.ds(q * 16, 16)] = lax.shift_right_logical(rc, 16)
            rcb[b, pl.ds(q * 16, 16)] = rc & lomask

    def gstart(b):
        pltpu.async_copy(hs_hbm.at[rcb.at[b]], gbuf.at[b], gsem)

    def gwait(b):
        pltpu.make_async_copy(hs_hbm.at[rcb.at[b]], gbuf.at[b], gsem).wait()

    def sstart(b):
        pltpu.async_copy(gbuf.at[b], acc.at[colb.at[b]], ssem, add=True)

    def swait(b):
        pltpu.make_async_copy(gbuf.at[b], acc.at[colb.at[b]], ssem).wait()

    istart(0, 0)
    istart(1, 1)
    istart(2, 2)
    iwait(0)
    split(0)
    gstart(0)
    iwait(1)
    split(1)
    gstart(1)

    def ring(j0, _):
        for b in range(NBUF):
            j = j0 + b
            gwait(b)
            sstart(b)

            @pl.when(jnp.logical_and(j >= 1, j <= steps - 3))
            def _():
                swait((b + 2) % NBUF)

            @pl.when(j <= steps - 4)
            def _():
                istart(j + 3, b)

            @pl.when(j <= steps - 3)
            def _():
                bn = (b + 2) % NBUF
                iwait(bn)
                split(bn)
                gstart(bn)
        return 0

    lax.fori_loop(0, steps // NBUF, lambda i, x: ring(i * NBUF, x), 0)
    for b in range(NBUF):
        swait(b)
    plsc.subcore_barrier()
    pltpu.sync_copy(acc.at[pl.ds(s * RPT, RPT)],
                    out_hbm.at[c, pl.ds(s * RPT, RPT)])


_agg_kernel = functools.partial(
    pl.kernel,
    out_type=jax.ShapeDtypeStruct((NC, PADN, DF), jnp.float32),
    mesh=_MESH,
    scratch_types=[
        pltpu.VMEM((NBUF, KE), jnp.int32),
        pltpu.VMEM((NBUF, KE), jnp.int32),
        pltpu.VMEM((NBUF, KE, DF), jnp.float32),
        pltpu.VMEM_SHARED((PADN, DF), jnp.float32),
        pltpu.SemaphoreType.DMA,
        pltpu.SemaphoreType.DMA,
        pltpu.SemaphoreType.DMA,
    ],
    compiler_params=pltpu.CompilerParams(needs_layout_passes=False),
)(_agg_body)


def _prep_body(x_ref, degp_ref, dinv_ref, xs_ref):
    deg = jnp.sum(degp_ref[...], axis=0)
    dinv = jnp.where(deg > 0, lax.rsqrt(jnp.maximum(deg, 1e-12)), 0.0)
    dinv_ref[...] = dinv[None, :]
    xs_ref[...] = x_ref[...] * dinv[:NN][:, None]


def _mid_body(p_ref, dinv_ref, w_ref, b_ref, hs_ref):
    dinv = dinv_ref[0, :NN]
    agg = (p_ref[0, :NN, :] + p_ref[1, :NN, :]) * dinv[:, None]
    h = jnp.dot(agg, w_ref[...], preferred_element_type=jnp.float32)
    h = h + b_ref[0, :][None, :]
    h = h * (1.0 / (1.0 + jnp.exp(-h)))
    hs_ref[...] = h * dinv[:, None]


def _final_body(p_ref, dinv_ref, w_ref, b_ref, out_ref):
    dinv = dinv_ref[0, :NN]
    agg = (p_ref[0, :NN, :] + p_ref[1, :NN, :]) * dinv[:, None]
    logits = jnp.dot(agg, w_ref[...], preferred_element_type=jnp.float32)
    logits = logits + b_ref[0, :][None, :]
    m = jnp.max(logits, axis=1, keepdims=True)
    z = logits - m
    lse = jnp.log(jnp.sum(jnp.exp(z), axis=1, keepdims=True))
    out_ref[...] = z - lse


def _tc_call(body, out_shape):
    return pl.pallas_call(body, out_shape=out_shape)


@jax.jit
def kernel(x, edge_index, W1, b1, W2, b2, W3, b3, W4, b4):
    loop = jnp.arange(NN, dtype=jnp.int32)
    row = jnp.concatenate(
        [edge_index[0], loop,
         jnp.zeros((EPAD - NN - edge_index.shape[1],), jnp.int32)])
    col = jnp.concatenate(
        [edge_index[1], loop,
         jnp.full((EPAD - NN - edge_index.shape[1],), PADN - 1, jnp.int32)])
    rc = row | (col << 16)
    degp = _deg_kernel(rc.reshape(NW, STEPSDEG, KE))

    dinv, xs = _tc_call(
        _prep_body,
        (jax.ShapeDtypeStruct((1, PADN), jnp.float32),
         jax.ShapeDtypeStruct((NN, DF), jnp.float32)),
    )(x, degp)

    h = xs
    for w, b in ((W1, b1), (W2, b2), (W3, b3)):
        p = _agg_kernel(h, rc)
        h = _tc_call(
            _mid_body, jax.ShapeDtypeStruct((NN, DF), jnp.float32),
        )(p, dinv, w, b[None, :])

    p = _agg_kernel(h, rc)
    out = _tc_call(
        _final_body,
        jax.ShapeDtypeStruct((NN, W4.shape[1]), jnp.float32),
    )(p, dinv, W4, b4[None, :])
    return out


# speedup vs baseline: 1.0420x; 1.0420x over previous
import functools

import jax
import jax.numpy as jnp
from jax import lax
from jax.experimental import pallas as pl
from jax.experimental.pallas import tpu as pltpu
from jax.experimental.pallas import tpu_sc as plsc

NN = 10000
DF = 128
NC = 2
NS = 16
NW = NC * NS
PADN = 10112
RPT = PADN // NS
KE = 128
EPAD = 331776
ET0 = 12672
ET1 = EPAD // NS - ET0
STEPS0 = ET0 // KE
STEPS1 = ET1 // KE
ETDEG = EPAD // NW
STEPSDEG = ETDEG // KE
NBUF = 3

_MESH = plsc.VectorSubcoreMesh(
    core_axis_name="c", subcore_axis_name="s", num_cores=NC, num_subcores=NS)


def _deg_body(rc_hbm, out_hbm, colv, degv):
    c = lax.axis_index("c")
    s = lax.axis_index("s")
    wid = c * NS + s
    pltpu.sync_copy(rc_hbm.at[wid], colv)

    zeros16 = jnp.zeros((16,), jnp.float32)
    ones16 = jnp.ones((16,), jnp.float32)

    def zero_step(i, _):
        degv[pl.ds(i * 16, 16)] = zeros16
        return 0

    lax.fori_loop(0, PADN // 16, zero_step, 0)

    def acc_step(r, _):
        for q in range(KE // 16):
            idx = lax.shift_right_logical(colv[r, pl.ds(q * 16, 16)], 16)
            plsc.addupdate_scatter(degv, [idx], ones16)
        return 0

    lax.fori_loop(0, STEPSDEG, acc_step, 0)
    pltpu.sync_copy(degv, out_hbm.at[wid])


_deg_kernel = functools.partial(
    pl.kernel,
    out_type=jax.ShapeDtypeStruct((NW, PADN), jnp.float32),
    mesh=_MESH,
    scratch_types=[
        pltpu.VMEM((STEPSDEG, KE), jnp.int32),
        pltpu.VMEM((PADN,), jnp.float32),
    ],
    compiler_params=pltpu.CompilerParams(needs_layout_passes=False),
)(_deg_body)


def _agg_body(hs_hbm, rc_hbm, out_hbm, rcb, colb, gbuf, acc,
              isem, gsem, ssem):
    c = lax.axis_index("c")
    s = lax.axis_index("s")
    steps = jnp.where(c == 0, STEPS0, STEPS1)
    base = jnp.where(c == 0, s * ET0, NS * ET0 + s * ET1)

    zeros16 = jnp.zeros((16,), jnp.float32)

    def zrow(i, _):
        for j in range(DF // 16):
            gbuf[0, i, pl.ds(j * 16, 16)] = zeros16
        return 0

    lax.fori_loop(0, KE, zrow, 0)
    for z in range((RPT + KE - 1) // KE):
        n = min(KE, RPT - z * KE)
        pltpu.sync_copy(gbuf.at[0, pl.ds(0, n)],
                        acc.at[pl.ds(s * RPT + z * KE, n)])
    plsc.subcore_barrier()

    lomask = jnp.full((16,), 65535, jnp.int32)

    def istart(j, b):
        off = base + j * KE
        pltpu.async_copy(rc_hbm.at[pl.ds(off, KE)], rcb.at[b], isem)

    def iwait(b):
        pltpu.make_async_copy(rc_hbm.at[pl.ds(0, KE)], rcb.at[b],
                              isem).wait()

    def split(b):
        for q in range(KE // 16):
            rc = rcb[b, pl.ds(q * 16, 16)]
            colb[b, pl.ds(q * 16, 16)] = lax.shift_right_logical(rc, 16)
            rcb[b, pl.ds(q * 16, 16)] = rc & lomask

    def gstart(b):
        pltpu.async_copy(hs_hbm.at[rcb.at[b]], gbuf.at[b], gsem)

    def gwait(b):
        pltpu.make_async_copy(hs_hbm.at[rcb.at[b]], gbuf.at[b], gsem).wait()

    def sstart(b):
        pltpu.async_copy(gbuf.at[b], acc.at[colb.at[b]], ssem, add=True)

    def swait(b):
        pltpu.make_async_copy(gbuf.at[b], acc.at[colb.at[b]], ssem).wait()

    istart(0, 0)
    istart(1, 1)
    istart(2, 2)
    iwait(0)
    split(0)
    gstart(0)
    iwait(1)
    split(1)
    gstart(1)

    def ring(j0, _):
        for b in range(NBUF):
            j = j0 + b
            gwait(b)
            sstart(b)

            @pl.when(jnp.logical_and(j >= 1, j <= steps - 3))
            def _():
                swait((b + 2) % NBUF)

            @pl.when(j <= steps - 4)
            def _():
                istart(j + 3, b)

            @pl.when(j <= steps - 3)
            def _():
                bn = (b + 2) % NBUF
                iwait(bn)
                split(bn)
                gstart(bn)
        return 0

    lax.fori_loop(0, steps // NBUF, lambda i, x: ring(i * NBUF, x), 0)
    for b in range(NBUF):
        swait(b)
    plsc.subcore_barrier()
    pltpu.sync_copy(acc.at[pl.ds(s * RPT, RPT)],
                    out_hbm.at[c, pl.ds(s * RPT, RPT)])


_agg_kernel = functools.partial(
    pl.kernel,
    out_type=jax.ShapeDtypeStruct((NC, PADN, DF), jnp.float32),
    mesh=_MESH,
    scratch_types=[
        pltpu.VMEM((NBUF, KE), jnp.int32),
        pltpu.VMEM((NBUF, KE), jnp.int32),
        pltpu.VMEM((NBUF, KE, DF), jnp.float32),
        pltpu.VMEM_SHARED((PADN, DF), jnp.float32),
        pltpu.SemaphoreType.DMA,
        pltpu.SemaphoreType.DMA,
        pltpu.SemaphoreType.DMA,
    ],
    compiler_params=pltpu.CompilerParams(needs_layout_passes=False),
)(_agg_body)


def _prep_body(x_ref, degp_ref, dinv_ref, xs_ref):
    deg = jnp.sum(degp_ref[...], axis=0)
    dinv = jnp.where(deg > 0, lax.rsqrt(jnp.maximum(deg, 1e-12)), 0.0)
    dinv_ref[...] = dinv[None, :]
    xs_ref[...] = x_ref[...] * dinv[:NN][:, None]


def _mid_body(p_ref, dinv_ref, w_ref, b_ref, hs_ref):
    dinv = dinv_ref[0, :NN]
    agg = (p_ref[0, :NN, :] + p_ref[1, :NN, :]) * dinv[:, None]
    h = jnp.dot(agg, w_ref[...], preferred_element_type=jnp.float32)
    h = h + b_ref[0, :][None, :]
    h = h * (1.0 / (1.0 + jnp.exp(-h)))
    hs_ref[...] = h * dinv[:, None]


def _final_body(p_ref, dinv_ref, w_ref, b_ref, out_ref):
    dinv = dinv_ref[0, :NN]
    agg = (p_ref[0, :NN, :] + p_ref[1, :NN, :]) * dinv[:, None]
    logits = jnp.dot(agg, w_ref[...], preferred_element_type=jnp.float32)
    logits = logits + b_ref[0, :][None, :]
    m = jnp.max(logits, axis=1, keepdims=True)
    z = logits - m
    lse = jnp.log(jnp.sum(jnp.exp(z), axis=1, keepdims=True))
    out_ref[...] = z - lse


def _tc_call(body, out_shape):
    return pl.pallas_call(body, out_shape=out_shape)


@jax.jit
def kernel(x, edge_index, W1, b1, W2, b2, W3, b3, W4, b4):
    loop = jnp.arange(NN, dtype=jnp.int32)
    row = jnp.concatenate(
        [edge_index[0], loop,
         jnp.zeros((EPAD - NN - edge_index.shape[1],), jnp.int32)])
    col = jnp.concatenate(
        [edge_index[1], loop,
         jnp.full((EPAD - NN - edge_index.shape[1],), PADN - 1, jnp.int32)])
    rc = row | (col << 16)
    degp = _deg_kernel(rc.reshape(NW, STEPSDEG, KE))

    dinv, xs = _tc_call(
        _prep_body,
        (jax.ShapeDtypeStruct((1, PADN), jnp.float32),
         jax.ShapeDtypeStruct((NN, DF), jnp.float32)),
    )(x, degp)

    h = xs
    for w, b in ((W1, b1), (W2, b2), (W3, b3)):
        p = _agg_kernel(h, rc)
        h = _tc_call(
            _mid_body, jax.ShapeDtypeStruct((NN, DF), jnp.float32),
        )(p, dinv, w, b[None, :])

    p = _agg_kernel(h, rc)
    out = _tc_call(
        _final_body,
        jax.ShapeDtypeStruct((NN, W4.shape[1]), jnp.float32),
    )(p, dinv, W4, b4[None, :])
    return out


# speedup vs baseline: 1.0527x; 1.0103x over previous
import functools

import jax
import jax.numpy as jnp
from jax import lax
from jax.experimental import pallas as pl
from jax.experimental.pallas import tpu as pltpu
from jax.experimental.pallas import tpu_sc as plsc

NN = 10000
DF = 128
NC = 2
NS = 16
NW = NC * NS
PADN = 10112
RPT = PADN // NS
KE = 128
EPAD = 331776
ET0 = 13056
ET1 = EPAD // NS - ET0
STEPS0 = ET0 // KE
STEPS1 = ET1 // KE
ETDEG = EPAD // NW
STEPSDEG = ETDEG // KE
NBUF = 3

_MESH = plsc.VectorSubcoreMesh(
    core_axis_name="c", subcore_axis_name="s", num_cores=NC, num_subcores=NS)


def _deg_body(rc_hbm, out_hbm, colv, degv):
    c = lax.axis_index("c")
    s = lax.axis_index("s")
    wid = c * NS + s
    pltpu.sync_copy(rc_hbm.at[wid], colv)

    zeros16 = jnp.zeros((16,), jnp.float32)
    ones16 = jnp.ones((16,), jnp.float32)

    def zero_step(i, _):
        degv[pl.ds(i * 16, 16)] = zeros16
        return 0

    lax.fori_loop(0, PADN // 16, zero_step, 0)

    def acc_step(r, _):
        for q in range(KE // 16):
            idx = lax.shift_right_logical(colv[r, pl.ds(q * 16, 16)], 16)
            plsc.addupdate_scatter(degv, [idx], ones16)
        return 0

    lax.fori_loop(0, STEPSDEG, acc_step, 0)
    pltpu.sync_copy(degv, out_hbm.at[wid])


_deg_kernel = functools.partial(
    pl.kernel,
    out_type=jax.ShapeDtypeStruct((NW, PADN), jnp.float32),
    mesh=_MESH,
    scratch_types=[
        pltpu.VMEM((STEPSDEG, KE), jnp.int32),
        pltpu.VMEM((PADN,), jnp.float32),
    ],
    compiler_params=pltpu.CompilerParams(needs_layout_passes=False),
)(_deg_body)


def _agg_body(hs_hbm, rc_hbm, out_hbm, rcb, colb, gbuf, acc,
              isem, gsem, ssem):
    c = lax.axis_index("c")
    s = lax.axis_index("s")
    steps = jnp.where(c == 0, STEPS0, STEPS1)
    base = jnp.where(c == 0, s * ET0, NS * ET0 + s * ET1)

    zeros16 = jnp.zeros((16,), jnp.float32)

    def zrow(i, _):
        for j in range(DF // 16):
            gbuf[0, i, pl.ds(j * 16, 16)] = zeros16
        return 0

    lax.fori_loop(0, KE, zrow, 0)
    for z in range((RPT + KE - 1) // KE):
        n = min(KE, RPT - z * KE)
        pltpu.sync_copy(gbuf.at[0, pl.ds(0, n)],
                        acc.at[pl.ds(s * RPT + z * KE, n)])
    plsc.subcore_barrier()

    lomask = jnp.full((16,), 65535, jnp.int32)

    def istart(j, b):
        off = base + j * KE
        pltpu.async_copy(rc_hbm.at[pl.ds(off, KE)], rcb.at[b], isem)

    def iwait(b):
        pltpu.make_async_copy(rc_hbm.at[pl.ds(0, KE)], rcb.at[b],
                              isem).wait()

    def split(b):
        for q in range(KE // 16):
            rc = rcb[b, pl.ds(q * 16, 16)]
            colb[b, pl.ds(q * 16, 16)] = lax.shift_right_logical(rc, 16)
            rcb[b, pl.ds(q * 16, 16)] = rc & lomask

    def gstart(b):
        pltpu.async_copy(hs_hbm.at[rcb.at[b]], gbuf.at[b], gsem)

    def gwait(b):
        pltpu.make_async_copy(hs_hbm.at[rcb.at[b]], gbuf.at[b], gsem).wait()

    def sstart(b):
        pltpu.async_copy(gbuf.at[b], acc.at[colb.at[b]], ssem, add=True)

    def swait(b):
        pltpu.make_async_copy(gbuf.at[b], acc.at[colb.at[b]], ssem).wait()

    istart(0, 0)
    istart(1, 1)
    istart(2, 2)
    iwait(0)
    split(0)
    gstart(0)
    iwait(1)
    split(1)
    gstart(1)

    def ring(j0, _):
        for b in range(NBUF):
            j = j0 + b
            gwait(b)
            sstart(b)

            @pl.when(jnp.logical_and(j >= 1, j <= steps - 3))
            def _():
                swait((b + 2) % NBUF)

            @pl.when(j <= steps - 4)
            def _():
                istart(j + 3, b)

            @pl.when(j <= steps - 3)
            def _():
                bn = (b + 2) % NBUF
                iwait(bn)
                split(bn)
                gstart(bn)
        return 0

    lax.fori_loop(0, steps // NBUF, lambda i, x: ring(i * NBUF, x), 0)
    for b in range(NBUF):
        swait(b)
    plsc.subcore_barrier()
    pltpu.sync_copy(acc.at[pl.ds(s * RPT, RPT)],
                    out_hbm.at[c, pl.ds(s * RPT, RPT)])


_agg_kernel = functools.partial(
    pl.kernel,
    out_type=jax.ShapeDtypeStruct((NC, PADN, DF), jnp.float32),
    mesh=_MESH,
    scratch_types=[
        pltpu.VMEM((NBUF, KE), jnp.int32),
        pltpu.VMEM((NBUF, KE), jnp.int32),
        pltpu.VMEM((NBUF, KE, DF), jnp.float32),
        pltpu.VMEM_SHARED((PADN, DF), jnp.float32),
        pltpu.SemaphoreType.DMA,
        pltpu.SemaphoreType.DMA,
        pltpu.SemaphoreType.DMA,
    ],
    compiler_params=pltpu.CompilerParams(needs_layout_passes=False),
)(_agg_body)


def _prep_body(x_ref, degp_ref, dinv_ref, xs_ref):
    deg = jnp.sum(degp_ref[...], axis=0)
    dinv = jnp.where(deg > 0, lax.rsqrt(jnp.maximum(deg, 1e-12)), 0.0)
    dinv_ref[...] = dinv[None, :]
    xs_ref[...] = x_ref[...] * dinv[:NN][:, None]


def _mid_body(p_ref, dinv_ref, w_ref, b_ref, hs_ref):
    dinv = dinv_ref[0, :NN]
    agg = (p_ref[0, :NN, :] + p_ref[1, :NN, :]) * dinv[:, None]
    h = jnp.dot(agg, w_ref[...], preferred_element_type=jnp.float32)
    h = h + b_ref[0, :][None, :]
    h = h * (1.0 / (1.0 + jnp.exp(-h)))
    hs_ref[...] = h * dinv[:, None]


def _final_body(p_ref, dinv_ref, w_ref, b_ref, out_ref):
    dinv = dinv_ref[0, :NN]
    agg = (p_ref[0, :NN, :] + p_ref[1, :NN, :]) * dinv[:, None]
    logits = jnp.dot(agg, w_ref[...], preferred_element_type=jnp.float32)
    logits = logits + b_ref[0, :][None, :]
    m = jnp.max(logits, axis=1, keepdims=True)
    z = logits - m
    lse = jnp.log(jnp.sum(jnp.exp(z), axis=1, keepdims=True))
    out_ref[...] = z - lse


def _tc_call(body, out_shape):
    return pl.pallas_call(body, out_shape=out_shape)


@jax.jit
def kernel(x, edge_index, W1, b1, W2, b2, W3, b3, W4, b4):
    loop = jnp.arange(NN, dtype=jnp.int32)
    row = jnp.concatenate(
        [edge_index[0], loop,
         jnp.zeros((EPAD - NN - edge_index.shape[1],), jnp.int32)])
    col = jnp.concatenate(
        [edge_index[1], loop,
         jnp.full((EPAD - NN - edge_index.shape[1],), PADN - 1, jnp.int32)])
    rc = row | (col << 16)
    degp = _deg_kernel(rc.reshape(NW, STEPSDEG, KE))

    dinv, xs = _tc_call(
        _prep_body,
        (jax.ShapeDtypeStruct((1, PADN), jnp.float32),
         jax.ShapeDtypeStruct((NN, DF), jnp.float32)),
    )(x, degp)

    h = xs
    for w, b in ((W1, b1), (W2, b2), (W3, b3)):
        p = _agg_kernel(h, rc)
        h = _tc_call(
            _mid_body, jax.ShapeDtypeStruct((NN, DF), jnp.float32),
        )(p, dinv, w, b[None, :])

    p = _agg_kernel(h, rc)
    out = _tc_call(
        _final_body,
        jax.ShapeDtypeStruct((NN, W4.shape[1]), jnp.float32),
    )(p, dinv, W4, b4[None, :])
    return out


# speedup vs baseline: 1.0701x; 1.0165x over previous
import functools

import jax
import jax.numpy as jnp
from jax import lax
from jax.experimental import pallas as pl
from jax.experimental.pallas import tpu as pltpu
from jax.experimental.pallas import tpu_sc as plsc

NN = 10000
DF = 128
NC = 2
NS = 16
NW = NC * NS
PADN = 10112
RPT = PADN // NS
KE = 128
EPAD = 331776
ET0 = 13440
ET1 = EPAD // NS - ET0
STEPS0 = ET0 // KE
STEPS1 = ET1 // KE
ETDEG = EPAD // NW
STEPSDEG = ETDEG // KE
NBUF = 3

_MESH = plsc.VectorSubcoreMesh(
    core_axis_name="c", subcore_axis_name="s", num_cores=NC, num_subcores=NS)


def _deg_body(rc_hbm, out_hbm, colv, degv):
    c = lax.axis_index("c")
    s = lax.axis_index("s")
    wid = c * NS + s
    pltpu.sync_copy(rc_hbm.at[wid], colv)

    zeros16 = jnp.zeros((16,), jnp.float32)
    ones16 = jnp.ones((16,), jnp.float32)

    def zero_step(i, _):
        degv[pl.ds(i * 16, 16)] = zeros16
        return 0

    lax.fori_loop(0, PADN // 16, zero_step, 0)

    def acc_step(r, _):
        for q in range(KE // 16):
            idx = lax.shift_right_logical(colv[r, pl.ds(q * 16, 16)], 16)
            plsc.addupdate_scatter(degv, [idx], ones16)
        return 0

    lax.fori_loop(0, STEPSDEG, acc_step, 0)
    pltpu.sync_copy(degv, out_hbm.at[wid])


_deg_kernel = functools.partial(
    pl.kernel,
    out_type=jax.ShapeDtypeStruct((NW, PADN), jnp.float32),
    mesh=_MESH,
    scratch_types=[
        pltpu.VMEM((STEPSDEG, KE), jnp.int32),
        pltpu.VMEM((PADN,), jnp.float32),
    ],
    compiler_params=pltpu.CompilerParams(needs_layout_passes=False),
)(_deg_body)


def _agg_body(hs_hbm, rc_hbm, out_hbm, rcb, colb, gbuf, acc,
              isem, gsem, ssem):
    c = lax.axis_index("c")
    s = lax.axis_index("s")
    steps = jnp.where(c == 0, STEPS0, STEPS1)
    base = jnp.where(c == 0, s * ET0, NS * ET0 + s * ET1)

    zeros16 = jnp.zeros((16,), jnp.float32)

    def zrow(i, _):
        for j in range(DF // 16):
            gbuf[0, i, pl.ds(j * 16, 16)] = zeros16
        return 0

    lax.fori_loop(0, KE, zrow, 0)
    for z in range((RPT + KE - 1) // KE):
        n = min(KE, RPT - z * KE)
        pltpu.sync_copy(gbuf.at[0, pl.ds(0, n)],
                        acc.at[pl.ds(s * RPT + z * KE, n)])
    plsc.subcore_barrier()

    lomask = jnp.full((16,), 65535, jnp.int32)

    def istart(j, b):
        off = base + j * KE
        pltpu.async_copy(rc_hbm.at[pl.ds(off, KE)], rcb.at[b], isem)

    def iwait(b):
        pltpu.make_async_copy(rc_hbm.at[pl.ds(0, KE)], rcb.at[b],
                              isem).wait()

    def split(b):
        for q in range(KE // 16):
            rc = rcb[b, pl.ds(q * 16, 16)]
            colb[b, pl.ds(q * 16, 16)] = lax.shift_right_logical(rc, 16)
            rcb[b, pl.ds(q * 16, 16)] = rc & lomask

    def gstart(b):
        pltpu.async_copy(hs_hbm.at[rcb.at[b]], gbuf.at[b], gsem)

    def gwait(b):
        pltpu.make_async_copy(hs_hbm.at[rcb.at[b]], gbuf.at[b], gsem).wait()

    def sstart(b):
        pltpu.async_copy(gbuf.at[b], acc.at[colb.at[b]], ssem, add=True)

    def swait(b):
        pltpu.make_async_copy(gbuf.at[b], acc.at[colb.at[b]], ssem).wait()

    istart(0, 0)
    istart(1, 1)
    istart(2, 2)
    iwait(0)
    split(0)
    gstart(0)
    iwait(1)
    split(1)
    gstart(1)

    def ring(j0, _):
        for b in range(NBUF):
            j = j0 + b
            gwait(b)
            sstart(b)

            @pl.when(jnp.logical_and(j >= 1, j <= steps - 3))
            def _():
                swait((b + 2) % NBUF)

            @pl.when(j <= steps - 4)
            def _():
                istart(j + 3, b)

            @pl.when(j <= steps - 3)
            def _():
                bn = (b + 2) % NBUF
                iwait(bn)
                split(bn)
                gstart(bn)
        return 0

    lax.fori_loop(0, steps // NBUF, lambda i, x: ring(i * NBUF, x), 0)
    for b in range(NBUF):
        swait(b)
    plsc.subcore_barrier()
    pltpu.sync_copy(acc.at[pl.ds(s * RPT, RPT)],
                    out_hbm.at[c, pl.ds(s * RPT, RPT)])


_agg_kernel = functools.partial(
    pl.kernel,
    out_type=jax.ShapeDtypeStruct((NC, PADN, DF), jnp.float32),
    mesh=_MESH,
    scratch_types=[
        pltpu.VMEM((NBUF, KE), jnp.int32),
        pltpu.VMEM((NBUF, KE), jnp.int32),
        pltpu.VMEM((NBUF, KE, DF), jnp.float32),
        pltpu.VMEM_SHARED((PADN, DF), jnp.float32),
        pltpu.SemaphoreType.DMA,
        pltpu.SemaphoreType.DMA,
        pltpu.SemaphoreType.DMA,
    ],
    compiler_params=pltpu.CompilerParams(needs_layout_passes=False),
)(_agg_body)


def _prep_body(x_ref, degp_ref, dinv_ref, xs_ref):
    deg = jnp.sum(degp_ref[...], axis=0)
    dinv = jnp.where(deg > 0, lax.rsqrt(jnp.maximum(deg, 1e-12)), 0.0)
    dinv_ref[...] = dinv[None, :]
    xs_ref[...] = x_ref[...] * dinv[:NN][:, None]


def _mid_body(p_ref, dinv_ref, w_ref, b_ref, hs_ref):
    dinv = dinv_ref[0, :NN]
    agg = (p_ref[0, :NN, :] + p_ref[1, :NN, :]) * dinv[:, None]
    h = jnp.dot(agg, w_ref[...], preferred_element_type=jnp.float32)
    h = h + b_ref[0, :][None, :]
    h = h * (1.0 / (1.0 + jnp.exp(-h)))
    hs_ref[...] = h * dinv[:, None]


def _final_body(p_ref, dinv_ref, w_ref, b_ref, out_ref):
    dinv = dinv_ref[0, :NN]
    agg = (p_ref[0, :NN, :] + p_ref[1, :NN, :]) * dinv[:, None]
    logits = jnp.dot(agg, w_ref[...], preferred_element_type=jnp.float32)
    logits = logits + b_ref[0, :][None, :]
    m = jnp.max(logits, axis=1, keepdims=True)
    z = logits - m
    lse = jnp.log(jnp.sum(jnp.exp(z), axis=1, keepdims=True))
    out_ref[...] = z - lse


def _tc_call(body, out_shape):
    return pl.pallas_call(body, out_shape=out_shape)


@jax.jit
def kernel(x, edge_index, W1, b1, W2, b2, W3, b3, W4, b4):
    loop = jnp.arange(NN, dtype=jnp.int32)
    row = jnp.concatenate(
        [edge_index[0], loop,
         jnp.zeros((EPAD - NN - edge_index.shape[1],), jnp.int32)])
    col = jnp.concatenate(
        [edge_index[1], loop,
         jnp.full((EPAD - NN - edge_index.shape[1],), PADN - 1, jnp.int32)])
    rc = row | (col << 16)
    degp = _deg_kernel(rc.reshape(NW, STEPSDEG, KE))

    dinv, xs = _tc_call(
        _prep_body,
        (jax.ShapeDtypeStruct((1, PADN), jnp.float32),
         jax.ShapeDtypeStruct((NN, DF), jnp.float32)),
    )(x, degp)

    h = xs
    for w, b in ((W1, b1), (W2, b2), (W3, b3)):
        p = _agg_kernel(h, rc)
        h = _tc_call(
            _mid_body, jax.ShapeDtypeStruct((NN, DF), jnp.float32),
        )(p, dinv, w, b[None, :])

    p = _agg_kernel(h, rc)
    out = _tc_call(
        _final_body,
        jax.ShapeDtypeStruct((NN, W4.shape[1]), jnp.float32),
    )(p, dinv, W4, b4[None, :])
    return out


# speedup vs baseline: 1.0822x; 1.0114x over previous
import functools

import jax
import jax.numpy as jnp
from jax import lax
from jax.experimental import pallas as pl
from jax.experimental.pallas import tpu as pltpu
from jax.experimental.pallas import tpu_sc as plsc

NN = 10000
DF = 128
NC = 2
NS = 16
NW = NC * NS
PADN = 10112
RPT = PADN // NS
KE = 128
EPAD = 331776
ET0 = 13824
ET1 = EPAD // NS - ET0
STEPS0 = ET0 // KE
STEPS1 = ET1 // KE
ETDEG = EPAD // NW
STEPSDEG = ETDEG // KE
NBUF = 3

_MESH = plsc.VectorSubcoreMesh(
    core_axis_name="c", subcore_axis_name="s", num_cores=NC, num_subcores=NS)


def _deg_body(rc_hbm, out_hbm, colv, degv):
    c = lax.axis_index("c")
    s = lax.axis_index("s")
    wid = c * NS + s
    pltpu.sync_copy(rc_hbm.at[wid], colv)

    zeros16 = jnp.zeros((16,), jnp.float32)
    ones16 = jnp.ones((16,), jnp.float32)

    def zero_step(i, _):
        degv[pl.ds(i * 16, 16)] = zeros16
        return 0

    lax.fori_loop(0, PADN // 16, zero_step, 0)

    def acc_step(r, _):
        for q in range(KE // 16):
            idx = lax.shift_right_logical(colv[r, pl.ds(q * 16, 16)], 16)
            plsc.addupdate_scatter(degv, [idx], ones16)
        return 0

    lax.fori_loop(0, STEPSDEG, acc_step, 0)
    pltpu.sync_copy(degv, out_hbm.at[wid])


_deg_kernel = functools.partial(
    pl.kernel,
    out_type=jax.ShapeDtypeStruct((NW, PADN), jnp.float32),
    mesh=_MESH,
    scratch_types=[
        pltpu.VMEM((STEPSDEG, KE), jnp.int32),
        pltpu.VMEM((PADN,), jnp.float32),
    ],
    compiler_params=pltpu.CompilerParams(needs_layout_passes=False),
)(_deg_body)


def _agg_body(hs_hbm, rc_hbm, out_hbm, rcb, colb, gbuf, acc,
              isem, gsem, ssem):
    c = lax.axis_index("c")
    s = lax.axis_index("s")
    steps = jnp.where(c == 0, STEPS0, STEPS1)
    base = jnp.where(c == 0, s * ET0, NS * ET0 + s * ET1)

    zeros16 = jnp.zeros((16,), jnp.float32)

    def zrow(i, _):
        for j in range(DF // 16):
            gbuf[0, i, pl.ds(j * 16, 16)] = zeros16
        return 0

    lax.fori_loop(0, KE, zrow, 0)
    for z in range((RPT + KE - 1) // KE):
        n = min(KE, RPT - z * KE)
        pltpu.sync_copy(gbuf.at[0, pl.ds(0, n)],
                        acc.at[pl.ds(s * RPT + z * KE, n)])
    plsc.subcore_barrier()

    lomask = jnp.full((16,), 65535, jnp.int32)

    def istart(j, b):
        off = base + j * KE
        pltpu.async_copy(rc_hbm.at[pl.ds(off, KE)], rcb.at[b], isem)

    def iwait(b):
        pltpu.make_async_copy(rc_hbm.at[pl.ds(0, KE)], rcb.at[b],
                              isem).wait()

    def split(b):
        for q in range(KE // 16):
            rc = rcb[b, pl.ds(q * 16, 16)]
            colb[b, pl.ds(q * 16, 16)] = lax.shift_right_logical(rc, 16)
            rcb[b, pl.ds(q * 16, 16)] = rc & lomask

    def gstart(b):
        pltpu.async_copy(hs_hbm.at[rcb.at[b]], gbuf.at[b], gsem)

    def gwait(b):
        pltpu.make_async_copy(hs_hbm.at[rcb.at[b]], gbuf.at[b], gsem).wait()

    def sstart(b):
        pltpu.async_copy(gbuf.at[b], acc.at[colb.at[b]], ssem, add=True)

    def swait(b):
        pltpu.make_async_copy(gbuf.at[b], acc.at[colb.at[b]], ssem).wait()

    istart(0, 0)
    istart(1, 1)
    istart(2, 2)
    iwait(0)
    split(0)
    gstart(0)
    iwait(1)
    split(1)
    gstart(1)

    def ring(j0, _):
        for b in range(NBUF):
            j = j0 + b
            gwait(b)
            sstart(b)

            @pl.when(jnp.logical_and(j >= 1, j <= steps - 3))
            def _():
                swait((b + 2) % NBUF)

            @pl.when(j <= steps - 4)
            def _():
                istart(j + 3, b)

            @pl.when(j <= steps - 3)
            def _():
                bn = (b + 2) % NBUF
                iwait(bn)
                split(bn)
                gstart(bn)
        return 0

    lax.fori_loop(0, steps // NBUF, lambda i, x: ring(i * NBUF, x), 0)
    for b in range(NBUF):
        swait(b)
    plsc.subcore_barrier()
    pltpu.sync_copy(acc.at[pl.ds(s * RPT, RPT)],
                    out_hbm.at[c, pl.ds(s * RPT, RPT)])


_agg_kernel = functools.partial(
    pl.kernel,
    out_type=jax.ShapeDtypeStruct((NC, PADN, DF), jnp.float32),
    mesh=_MESH,
    scratch_types=[
        pltpu.VMEM((NBUF, KE), jnp.int32),
        pltpu.VMEM((NBUF, KE), jnp.int32),
        pltpu.VMEM((NBUF, KE, DF), jnp.float32),
        pltpu.VMEM_SHARED((PADN, DF), jnp.float32),
        pltpu.SemaphoreType.DMA,
        pltpu.SemaphoreType.DMA,
        pltpu.SemaphoreType.DMA,
    ],
    compiler_params=pltpu.CompilerParams(needs_layout_passes=False),
)(_agg_body)


def _prep_body(x_ref, degp_ref, dinv_ref, xs_ref):
    deg = jnp.sum(degp_ref[...], axis=0)
    dinv = jnp.where(deg > 0, lax.rsqrt(jnp.maximum(deg, 1e-12)), 0.0)
    dinv_ref[...] = dinv[None, :]
    xs_ref[...] = x_ref[...] * dinv[:NN][:, None]


def _mid_body(p_ref, dinv_ref, w_ref, b_ref, hs_ref):
    dinv = dinv_ref[0, :NN]
    agg = (p_ref[0, :NN, :] + p_ref[1, :NN, :]) * dinv[:, None]
    h = jnp.dot(agg, w_ref[...], preferred_element_type=jnp.float32)
    h = h + b_ref[0, :][None, :]
    h = h * (1.0 / (1.0 + jnp.exp(-h)))
    hs_ref[...] = h * dinv[:, None]


def _final_body(p_ref, dinv_ref, w_ref, b_ref, out_ref):
    dinv = dinv_ref[0, :NN]
    agg = (p_ref[0, :NN, :] + p_ref[1, :NN, :]) * dinv[:, None]
    logits = jnp.dot(agg, w_ref[...], preferred_element_type=jnp.float32)
    logits = logits + b_ref[0, :][None, :]
    m = jnp.max(logits, axis=1, keepdims=True)
    z = logits - m
    lse = jnp.log(jnp.sum(jnp.exp(z), axis=1, keepdims=True))
    out_ref[...] = z - lse


def _tc_call(body, out_shape):
    return pl.pallas_call(body, out_shape=out_shape)


@jax.jit
def kernel(x, edge_index, W1, b1, W2, b2, W3, b3, W4, b4):
    loop = jnp.arange(NN, dtype=jnp.int32)
    row = jnp.concatenate(
        [edge_index[0], loop,
         jnp.zeros((EPAD - NN - edge_index.shape[1],), jnp.int32)])
    col = jnp.concatenate(
        [edge_index[1], loop,
         jnp.full((EPAD - NN - edge_index.shape[1],), PADN - 1, jnp.int32)])
    rc = row | (col << 16)
    degp = _deg_kernel(rc.reshape(NW, STEPSDEG, KE))

    dinv, xs = _tc_call(
        _prep_body,
        (jax.ShapeDtypeStruct((1, PADN), jnp.float32),
         jax.ShapeDtypeStruct((NN, DF), jnp.float32)),
    )(x, degp)

    h = xs
    for w, b in ((W1, b1), (W2, b2), (W3, b3)):
        p = _agg_kernel(h, rc)
        h = _tc_call(
            _mid_body, jax.ShapeDtypeStruct((NN, DF), jnp.float32),
        )(p, dinv, w, b[None, :])

    p = _agg_kernel(h, rc)
    out = _tc_call(
        _final_body,
        jax.ShapeDtypeStruct((NN, W4.shape[1]), jnp.float32),
    )(p, dinv, W4, b4[None, :])
    return out


# speedup vs baseline: 1.1073x; 1.0232x over previous
import functools

import jax
import jax.numpy as jnp
from jax import lax
from jax.experimental import pallas as pl
from jax.experimental.pallas import tpu as pltpu
from jax.experimental.pallas import tpu_sc as plsc

NN = 10000
DF = 128
NC = 2
NS = 16
NW = NC * NS
PADN = 10112
RPT = PADN // NS
KE = 128
EPAD = 331776
ET0 = 14592
ET1 = EPAD // NS - ET0
STEPS0 = ET0 // KE
STEPS1 = ET1 // KE
ETDEG = EPAD // NW
STEPSDEG = ETDEG // KE
NBUF = 3

_MESH = plsc.VectorSubcoreMesh(
    core_axis_name="c", subcore_axis_name="s", num_cores=NC, num_subcores=NS)


def _deg_body(rc_hbm, out_hbm, colv, degv):
    c = lax.axis_index("c")
    s = lax.axis_index("s")
    wid = c * NS + s
    pltpu.sync_copy(rc_hbm.at[wid], colv)

    zeros16 = jnp.zeros((16,), jnp.float32)
    ones16 = jnp.ones((16,), jnp.float32)

    def zero_step(i, _):
        degv[pl.ds(i * 16, 16)] = zeros16
        return 0

    lax.fori_loop(0, PADN // 16, zero_step, 0)

    def acc_step(r, _):
        for q in range(KE // 16):
            idx = lax.shift_right_logical(colv[r, pl.ds(q * 16, 16)], 16)
            plsc.addupdate_scatter(degv, [idx], ones16)
        return 0

    lax.fori_loop(0, STEPSDEG, acc_step, 0)
    pltpu.sync_copy(degv, out_hbm.at[wid])


_deg_kernel = functools.partial(
    pl.kernel,
    out_type=jax.ShapeDtypeStruct((NW, PADN), jnp.float32),
    mesh=_MESH,
    scratch_types=[
        pltpu.VMEM((STEPSDEG, KE), jnp.int32),
        pltpu.VMEM((PADN,), jnp.float32),
    ],
    compiler_params=pltpu.CompilerParams(needs_layout_passes=False),
)(_deg_body)


def _agg_body(hs_hbm, rc_hbm, out_hbm, rcb, colb, gbuf, acc,
              isem, gsem, ssem):
    c = lax.axis_index("c")
    s = lax.axis_index("s")
    steps = jnp.where(c == 0, STEPS0, STEPS1)
    base = jnp.where(c == 0, s * ET0, NS * ET0 + s * ET1)

    zeros16 = jnp.zeros((16,), jnp.float32)

    def zrow(i, _):
        for j in range(DF // 16):
            gbuf[0, i, pl.ds(j * 16, 16)] = zeros16
        return 0

    lax.fori_loop(0, KE, zrow, 0)
    for z in range((RPT + KE - 1) // KE):
        n = min(KE, RPT - z * KE)
        pltpu.sync_copy(gbuf.at[0, pl.ds(0, n)],
                        acc.at[pl.ds(s * RPT + z * KE, n)])
    plsc.subcore_barrier()

    lomask = jnp.full((16,), 65535, jnp.int32)

    def istart(j, b):
        off = base + j * KE
        pltpu.async_copy(rc_hbm.at[pl.ds(off, KE)], rcb.at[b], isem)

    def iwait(b):
        pltpu.make_async_copy(rc_hbm.at[pl.ds(0, KE)], rcb.at[b],
                              isem).wait()

    def split(b):
        for q in range(KE // 16):
            rc = rcb[b, pl.ds(q * 16, 16)]
            colb[b, pl.ds(q * 16, 16)] = lax.shift_right_logical(rc, 16)
            rcb[b, pl.ds(q * 16, 16)] = rc & lomask

    def gstart(b):
        pltpu.async_copy(hs_hbm.at[rcb.at[b]], gbuf.at[b], gsem)

    def gwait(b):
        pltpu.make_async_copy(hs_hbm.at[rcb.at[b]], gbuf.at[b], gsem).wait()

    def sstart(b):
        pltpu.async_copy(gbuf.at[b], acc.at[colb.at[b]], ssem, add=True)

    def swait(b):
        pltpu.make_async_copy(gbuf.at[b], acc.at[colb.at[b]], ssem).wait()

    istart(0, 0)
    istart(1, 1)
    istart(2, 2)
    iwait(0)
    split(0)
    gstart(0)
    iwait(1)
    split(1)
    gstart(1)

    def ring(j0, _):
        for b in range(NBUF):
            j = j0 + b
            gwait(b)
            sstart(b)

            @pl.when(jnp.logical_and(j >= 1, j <= steps - 3))
            def _():
                swait((b + 2) % NBUF)

            @pl.when(j <= steps - 4)
            def _():
                istart(j + 3, b)

            @pl.when(j <= steps - 3)
            def _():
                bn = (b + 2) % NBUF
                iwait(bn)
                split(bn)
                gstart(bn)
        return 0

    lax.fori_loop(0, steps // NBUF, lambda i, x: ring(i * NBUF, x), 0)
    for b in range(NBUF):
        swait(b)
    plsc.subcore_barrier()
    pltpu.sync_copy(acc.at[pl.ds(s * RPT, RPT)],
                    out_hbm.at[c, pl.ds(s * RPT, RPT)])


_agg_kernel = functools.partial(
    pl.kernel,
    out_type=jax.ShapeDtypeStruct((NC, PADN, DF), jnp.float32),
    mesh=_MESH,
    scratch_types=[
        pltpu.VMEM((NBUF, KE), jnp.int32),
        pltpu.VMEM((NBUF, KE), jnp.int32),
        pltpu.VMEM((NBUF, KE, DF), jnp.float32),
        pltpu.VMEM_SHARED((PADN, DF), jnp.float32),
        pltpu.SemaphoreType.DMA,
        pltpu.SemaphoreType.DMA,
        pltpu.SemaphoreType.DMA,
    ],
    compiler_params=pltpu.CompilerParams(needs_layout_passes=False),
)(_agg_body)


def _prep_body(x_ref, degp_ref, dinv_ref, xs_ref):
    deg = jnp.sum(degp_ref[...], axis=0)
    dinv = jnp.where(deg > 0, lax.rsqrt(jnp.maximum(deg, 1e-12)), 0.0)
    dinv_ref[...] = dinv[None, :]
    xs_ref[...] = x_ref[...] * dinv[:NN][:, None]


def _mid_body(p_ref, dinv_ref, w_ref, b_ref, hs_ref):
    dinv = dinv_ref[0, :NN]
    agg = (p_ref[0, :NN, :] + p_ref[1, :NN, :]) * dinv[:, None]
    h = jnp.dot(agg, w_ref[...], preferred_element_type=jnp.float32)
    h = h + b_ref[0, :][None, :]
    h = h * (1.0 / (1.0 + jnp.exp(-h)))
    hs_ref[...] = h * dinv[:, None]


def _final_body(p_ref, dinv_ref, w_ref, b_ref, out_ref):
    dinv = dinv_ref[0, :NN]
    agg = (p_ref[0, :NN, :] + p_ref[1, :NN, :]) * dinv[:, None]
    logits = jnp.dot(agg, w_ref[...], preferred_element_type=jnp.float32)
    logits = logits + b_ref[0, :][None, :]
    m = jnp.max(logits, axis=1, keepdims=True)
    z = logits - m
    lse = jnp.log(jnp.sum(jnp.exp(z), axis=1, keepdims=True))
    out_ref[...] = z - lse


def _tc_call(body, out_shape):
    return pl.pallas_call(body, out_shape=out_shape)


@jax.jit
def kernel(x, edge_index, W1, b1, W2, b2, W3, b3, W4, b4):
    loop = jnp.arange(NN, dtype=jnp.int32)
    row = jnp.concatenate(
        [edge_index[0], loop,
         jnp.zeros((EPAD - NN - edge_index.shape[1],), jnp.int32)])
    col = jnp.concatenate(
        [edge_index[1], loop,
         jnp.full((EPAD - NN - edge_index.shape[1],), PADN - 1, jnp.int32)])
    rc = row | (col << 16)
    degp = _deg_kernel(rc.reshape(NW, STEPSDEG, KE))

    dinv, xs = _tc_call(
        _prep_body,
        (jax.ShapeDtypeStruct((1, PADN), jnp.float32),
         jax.ShapeDtypeStruct((NN, DF), jnp.float32)),
    )(x, degp)

    h = xs
    for w, b in ((W1, b1), (W2, b2), (W3, b3)):
        p = _agg_kernel(h, rc)
        h = _tc_call(
            _mid_body, jax.ShapeDtypeStruct((NN, DF), jnp.float32),
        )(p, dinv, w, b[None, :])

    p = _agg_kernel(h, rc)
    out = _tc_call(
        _final_body,
        jax.ShapeDtypeStruct((NN, W4.shape[1]), jnp.float32),
    )(p, dinv, W4, b4[None, :])
    return out


# speedup vs baseline: 1.1432x; 1.0324x over previous
import functools

import jax
import jax.numpy as jnp
from jax import lax
from jax.experimental import pallas as pl
from jax.experimental.pallas import tpu as pltpu
from jax.experimental.pallas import tpu_sc as plsc

NN = 10000
DF = 128
NC = 2
NS = 16
NW = NC * NS
PADN = 10112
RPT = PADN // NS
KE = 128
EPAD = 331776
ET0 = 16896
ET1 = EPAD // NS - ET0
STEPS0 = ET0 // KE
STEPS1 = ET1 // KE
ETDEG = EPAD // NW
STEPSDEG = ETDEG // KE
NBUF = 3

_MESH = plsc.VectorSubcoreMesh(
    core_axis_name="c", subcore_axis_name="s", num_cores=NC, num_subcores=NS)


def _deg_body(rc_hbm, out_hbm, colv, degv):
    c = lax.axis_index("c")
    s = lax.axis_index("s")
    wid = c * NS + s
    pltpu.sync_copy(rc_hbm.at[wid], colv)

    zeros16 = jnp.zeros((16,), jnp.float32)
    ones16 = jnp.ones((16,), jnp.float32)

    def zero_step(i, _):
        degv[pl.ds(i * 16, 16)] = zeros16
        return 0

    lax.fori_loop(0, PADN // 16, zero_step, 0)

    def acc_step(r, _):
        for q in range(KE // 16):
            idx = lax.shift_right_logical(colv[r, pl.ds(q * 16, 16)], 16)
            plsc.addupdate_scatter(degv, [idx], ones16)
        return 0

    lax.fori_loop(0, STEPSDEG, acc_step, 0)
    pltpu.sync_copy(degv, out_hbm.at[wid])


_deg_kernel = functools.partial(
    pl.kernel,
    out_type=jax.ShapeDtypeStruct((NW, PADN), jnp.float32),
    mesh=_MESH,
    scratch_types=[
        pltpu.VMEM((STEPSDEG, KE), jnp.int32),
        pltpu.VMEM((PADN,), jnp.float32),
    ],
    compiler_params=pltpu.CompilerParams(needs_layout_passes=False),
)(_deg_body)


def _agg_body(hs_hbm, rc_hbm, out_hbm, rcb, colb, gbuf, acc,
              isem, gsem, ssem):
    c = lax.axis_index("c")
    s = lax.axis_index("s")
    steps = jnp.where(c == 0, STEPS0, STEPS1)
    base = jnp.where(c == 0, s * ET0, NS * ET0 + s * ET1)

    zeros16 = jnp.zeros((16,), jnp.float32)

    def zrow(i, _):
        for j in range(DF // 16):
            gbuf[0, i, pl.ds(j * 16, 16)] = zeros16
        return 0

    lax.fori_loop(0, KE, zrow, 0)
    for z in range((RPT + KE - 1) // KE):
        n = min(KE, RPT - z * KE)
        pltpu.sync_copy(gbuf.at[0, pl.ds(0, n)],
                        acc.at[pl.ds(s * RPT + z * KE, n)])
    plsc.subcore_barrier()

    lomask = jnp.full((16,), 65535, jnp.int32)

    def istart(j, b):
        off = base + j * KE
        pltpu.async_copy(rc_hbm.at[pl.ds(off, KE)], rcb.at[b], isem)

    def iwait(b):
        pltpu.make_async_copy(rc_hbm.at[pl.ds(0, KE)], rcb.at[b],
                              isem).wait()

    def split(b):
        for q in range(KE // 16):
            rc = rcb[b, pl.ds(q * 16, 16)]
            colb[b, pl.ds(q * 16, 16)] = lax.shift_right_logical(rc, 16)
            rcb[b, pl.ds(q * 16, 16)] = rc & lomask

    def gstart(b):
        pltpu.async_copy(hs_hbm.at[rcb.at[b]], gbuf.at[b], gsem)

    def gwait(b):
        pltpu.make_async_copy(hs_hbm.at[rcb.at[b]], gbuf.at[b], gsem).wait()

    def sstart(b):
        pltpu.async_copy(gbuf.at[b], acc.at[colb.at[b]], ssem, add=True)

    def swait(b):
        pltpu.make_async_copy(gbuf.at[b], acc.at[colb.at[b]], ssem).wait()

    istart(0, 0)
    istart(1, 1)
    istart(2, 2)
    iwait(0)
    split(0)
    gstart(0)
    iwait(1)
    split(1)
    gstart(1)

    def ring(j0, _):
        for b in range(NBUF):
            j = j0 + b
            gwait(b)
            sstart(b)

            @pl.when(jnp.logical_and(j >= 1, j <= steps - 3))
            def _():
                swait((b + 2) % NBUF)

            @pl.when(j <= steps - 4)
            def _():
                istart(j + 3, b)

            @pl.when(j <= steps - 3)
            def _():
                bn = (b + 2) % NBUF
                iwait(bn)
                split(bn)
                gstart(bn)
        return 0

    lax.fori_loop(0, steps // NBUF, lambda i, x: ring(i * NBUF, x), 0)
    for b in range(NBUF):
        swait(b)
    plsc.subcore_barrier()
    pltpu.sync_copy(acc.at[pl.ds(s * RPT, RPT)],
                    out_hbm.at[c, pl.ds(s * RPT, RPT)])


_agg_kernel = functools.partial(
    pl.kernel,
    out_type=jax.ShapeDtypeStruct((NC, PADN, DF), jnp.float32),
    mesh=_MESH,
    scratch_types=[
        pltpu.VMEM((NBUF, KE), jnp.int32),
        pltpu.VMEM((NBUF, KE), jnp.int32),
        pltpu.VMEM((NBUF, KE, DF), jnp.float32),
        pltpu.VMEM_SHARED((PADN, DF), jnp.float32),
        pltpu.SemaphoreType.DMA,
        pltpu.SemaphoreType.DMA,
        pltpu.SemaphoreType.DMA,
    ],
    compiler_params=pltpu.CompilerParams(needs_layout_passes=False),
)(_agg_body)


def _prep_body(x_ref, degp_ref, dinv_ref, xs_ref):
    deg = jnp.sum(degp_ref[...], axis=0)
    dinv = jnp.where(deg > 0, lax.rsqrt(jnp.maximum(deg, 1e-12)), 0.0)
    dinv_ref[...] = dinv[None, :]
    xs_ref[...] = x_ref[...] * dinv[:NN][:, None]


def _mid_body(p_ref, dinv_ref, w_ref, b_ref, hs_ref):
    dinv = dinv_ref[0, :NN]
    agg = (p_ref[0, :NN, :] + p_ref[1, :NN, :]) * dinv[:, None]
    h = jnp.dot(agg, w_ref[...], preferred_element_type=jnp.float32)
    h = h + b_ref[0, :][None, :]
    h = h * (1.0 / (1.0 + jnp.exp(-h)))
    hs_ref[...] = h * dinv[:, None]


def _final_body(p_ref, dinv_ref, w_ref, b_ref, out_ref):
    dinv = dinv_ref[0, :NN]
    agg = (p_ref[0, :NN, :] + p_ref[1, :NN, :]) * dinv[:, None]
    logits = jnp.dot(agg, w_ref[...], preferred_element_type=jnp.float32)
    logits = logits + b_ref[0, :][None, :]
    m = jnp.max(logits, axis=1, keepdims=True)
    z = logits - m
    lse = jnp.log(jnp.sum(jnp.exp(z), axis=1, keepdims=True))
    out_ref[...] = z - lse


def _tc_call(body, out_shape):
    return pl.pallas_call(body, out_shape=out_shape)


@jax.jit
def kernel(x, edge_index, W1, b1, W2, b2, W3, b3, W4, b4):
    loop = jnp.arange(NN, dtype=jnp.int32)
    row = jnp.concatenate(
        [edge_index[0], loop,
         jnp.zeros((EPAD - NN - edge_index.shape[1],), jnp.int32)])
    col = jnp.concatenate(
        [edge_index[1], loop,
         jnp.full((EPAD - NN - edge_index.shape[1],), PADN - 1, jnp.int32)])
    rc = row | (col << 16)
    degp = _deg_kernel(rc.reshape(NW, STEPSDEG, KE))

    dinv, xs = _tc_call(
        _prep_body,
        (jax.ShapeDtypeStruct((1, PADN), jnp.float32),
         jax.ShapeDtypeStruct((NN, DF), jnp.float32)),
    )(x, degp)

    h = xs
    for w, b in ((W1, b1), (W2, b2), (W3, b3)):
        p = _agg_kernel(h, rc)
        h = _tc_call(
            _mid_body, jax.ShapeDtypeStruct((NN, DF), jnp.float32),
        )(p, dinv, w, b[None, :])

    p = _agg_kernel(h, rc)
    out = _tc_call(
        _final_body,
        jax.ShapeDtypeStruct((NN, W4.shape[1]), jnp.float32),
    )(p, dinv, W4, b4[None, :])
    return out
